# baseline (speedup 1.0000x reference)
import functools

import jax
import jax.numpy as jnp
from jax import lax
from jax.experimental import pallas as pl
from jax.experimental.pallas import tpu as pltpu
from jax.experimental.pallas import tpu_sc as plsc

_B = 2
_C = 192
_H = 224
_W = 224
_WH = 112
_NBLK = _B * _H * 2

_NC = 2
_NS = 16
_NW = _NC * _NS
_BPW = _NBLK // _NW
_HPW = _H * _B // _NW
_XROWS = 24


def _body(x_hbm, s_hbm, out_hbm,
          bufo0, bufb0, bufo1, bufb1, bufxt,
          sem_l0, sem_l1, sem_s0, sem_s1, sem_x):
    bufo = (bufo0, bufo1)
    bufb = (bufb0, bufb1)
    sems_l = (sem_l0, sem_l1)
    sems_s = (sem_s0, sem_s1)
    wid = lax.axis_index("s") * _NC + lax.axis_index("c")
    base = wid * _BPW

    b0 = wid // (_NW // _B)
    h0 = (wid - b0 * (_NW // _B)) * _HPW
    h0a = jnp.minimum((h0 // 8) * 8, _H - _XROWS)
    h0a = pl.multiple_of(h0a, 8)
    pltpu.make_async_copy(
        x_hbm.at[b0, 0, pl.ds(h0a, _XROWS)], bufxt, sem_x).start()

    def coords(k):
        cid = base + k
        b = cid // (_H * 2)
        rem = cid - b * (_H * 2)
        h = rem // 2
        wh = (k % 2) * _WH
        return b, h, wh

    loads = {}
    stores = {}

    def start_loads(k):
        b, h, wh = coords(k)
        m = k % 2
        hs = [
            pltpu.make_async_copy(
                s_hbm.at[b, h, pl.ds(wh, _WH), pl.ds(0, 128)],
                bufo[m].at[:, pl.ds(0, 128)], sems_l[m]),
            pltpu.make_async_copy(
                s_hbm.at[b, h, pl.ds(wh, _WH), pl.ds(128, 128)],
                bufb[m], sems_l[m]),
        ]
        for hnd in hs:
            hnd.start()
        loads[k] = hs

    def merge(k):
        _, h, wh = coords(k)
        m = k % 2
        hr = h - h0a
        lane = lax.iota(jnp.int32, 16)

        for i in range(_WH // 16):
            vx = bufxt[hr, pl.ds(wh + 16 * i, 16)]
            plsc.store_scatter(bufo[m], [16 * i + lane, lane * 0], vx)

        def mbody(w, carry):
            for j in range(4):
                bufo[m][w, pl.ds(128 + 16 * j, 16)] = \
                    bufb[m][w, pl.ds(16 * j, 16)]
            return carry

        lax.fori_loop(0, _WH, mbody, 0)

    def start_store(k):
        b, h, wh = coords(k)
        m = k % 2
        h_st = pltpu.make_async_copy(
            bufo[m], out_hbm.at[b, h, pl.ds(wh, _WH)], sems_s[m])
        h_st.start()
        stores[k] = h_st

    pltpu.make_async_copy(
        x_hbm.at[b0, 0, pl.ds(h0a, _XROWS)], bufxt, sem_x).wait()
    start_loads(0)
    for k in range(_BPW):
        if k + 1 < _BPW:
            if k - 1 >= 0:
                stores[k - 1].wait()
            start_loads(k + 1)
        for hnd in loads[k]:
            hnd.wait()
        merge(k)
        start_store(k)
    stores[_BPW - 2].wait()
    stores[_BPW - 1].wait()


def kernel(x, shortcut_input):
    s_t = shortcut_input.transpose(0, 2, 3, 1)
    mesh = plsc.VectorSubcoreMesh(
        core_axis_name="c", subcore_axis_name="s",
        num_cores=_NC, num_subcores=_NS)
    run = functools.partial(
        pl.kernel,
        mesh=mesh,
        out_type=jax.ShapeDtypeStruct((_B, _H, _W, _C), jnp.float32),
        scratch_types=[
            pltpu.VMEM((_WH, _C), jnp.float32),
            pltpu.VMEM((_WH, 128), jnp.float32),
            pltpu.VMEM((_WH, _C), jnp.float32),
            pltpu.VMEM((_WH, 128), jnp.float32),
            pltpu.VMEM((_XROWS, _W), jnp.float32),
            pltpu.SemaphoreType.DMA,
            pltpu.SemaphoreType.DMA,
            pltpu.SemaphoreType.DMA,
            pltpu.SemaphoreType.DMA,
            pltpu.SemaphoreType.DMA,
        ],
        compiler_params=pltpu.CompilerParams(
            use_tc_tiling_on_sc=True, needs_layout_passes=False),
    )(_body)
    out_t = run(x, s_t)
    return out_t.transpose(0, 3, 1, 2)


# speedup vs baseline: 2.0597x; 2.0597x over previous
import functools

import jax
import jax.numpy as jnp
from jax import lax
from jax.experimental import pallas as pl
from jax.experimental.pallas import tpu as pltpu
from jax.experimental.pallas import tpu_sc as plsc

_B = 2
_C = 192
_H = 224
_W = 224
_NIMG = _B * _C

_NC = 2
_NS = 16
_NW = _NC * _NS
_IPW = _NIMG // _NW


def _body(x_hbm, s_hbm, out_hbm, buf0, buf1, sem_l0, sem_l1, sem_s0, sem_s1):
    bufs = (buf0, buf1)
    sems_l = (sem_l0, sem_l1)
    sems_s = (sem_s0, sem_s1)
    wid = lax.axis_index("s") * _NC + lax.axis_index("c")
    base = wid * _IPW

    def coords(k):
        r = base + k
        b = jnp.where(r >= _C, 1, 0)
        c = r - b * _C
        return b, c

    def start_load(k):
        b, c = coords(k)
        is_x = c == 0

        @pl.when(is_x)
        def _():
            pltpu.make_async_copy(
                x_hbm.at[b, 0], bufs[k % 2], sems_l[k % 2]).start()

        @pl.when(jnp.logical_not(is_x))
        def _():
            pltpu.make_async_copy(
                s_hbm.at[b, c], bufs[k % 2], sems_l[k % 2]).start()

    def wait_load(k):
        pltpu.make_async_copy(
            s_hbm.at[0, 0], bufs[k % 2], sems_l[k % 2]).wait()

    stores = {}

    def start_store(k):
        b, c = coords(k)
        h = pltpu.make_async_copy(
            bufs[k % 2], out_hbm.at[b, c], sems_s[k % 2])
        h.start()
        stores[k] = h

    start_load(0)
    for k in range(_IPW):
        if k + 1 < _IPW:
            if k - 1 >= 0:
                stores[k - 1].wait()
            start_load(k + 1)
        wait_load(k)
        start_store(k)
    stores[_IPW - 2].wait()
    stores[_IPW - 1].wait()


def kernel(x, shortcut_input):
    s01 = shortcut_input[:, 0:_C]
    mesh = plsc.VectorSubcoreMesh(
        core_axis_name="c", subcore_axis_name="s",
        num_cores=_NC, num_subcores=_NS)
    run = functools.partial(
        pl.kernel,
        mesh=mesh,
        out_type=jax.ShapeDtypeStruct((_B, _C, _H, _W), jnp.float32),
        scratch_types=[
            pltpu.VMEM((_H, _W), jnp.float32),
            pltpu.VMEM((_H, _W), jnp.float32),
            pltpu.SemaphoreType.DMA,
            pltpu.SemaphoreType.DMA,
            pltpu.SemaphoreType.DMA,
            pltpu.SemaphoreType.DMA,
        ],
        compiler_params=pltpu.CompilerParams(use_tc_tiling_on_sc=True),
    )(_body)
    return run(x, s01)
